# baseline (speedup 1.0000x reference)
import jax
import jax.numpy as jnp
from jax import lax
from jax.experimental import pallas as pl

_HIDDEN = 2048
_EXPERTS = 16
_THRESH = 0.2
_BM = 512


def _router_block(h_ref, w_ref, b_ref, idx_ref, mask_ref):
    h = h_ref[...]
    w = w_ref[...]
    logits = lax.dot_general(
        h, w, (((1,), (1,)), ((), ())), preferred_element_type=jnp.float32
    ) + b_ref[...]
    bm = logits.shape[0]
    iota = lax.broadcasted_iota(jnp.int32, (bm, _EXPERTS), 1)
    m1 = jnp.max(logits, axis=1, keepdims=True)
    i1 = jnp.min(jnp.where(logits == m1, iota, _EXPERTS), axis=1, keepdims=True)
    masked = jnp.where(iota == i1, -jnp.inf, logits)
    m2 = jnp.max(masked, axis=1, keepdims=True)
    i2 = jnp.min(jnp.where(masked == m2, iota, _EXPERTS), axis=1, keepdims=True)
    idx_ref[...] = jnp.concatenate([i1, i2], axis=1)
    mask_ref[...] = (jnp.concatenate([m1, m2], axis=1) > _THRESH).astype(jnp.float32)


def kernel(hidden_states, W, b):
    tokens = hidden_states.shape[0]
    grid = (tokens // _BM,)
    b2 = b.reshape(1, _EXPERTS)
    out_shapes = (
        jax.ShapeDtypeStruct((tokens, 2), jnp.int32),
        jax.ShapeDtypeStruct((tokens, 2), jnp.float32),
    )
    idx, mask = pl.pallas_call(
        _router_block,
        grid=grid,
        in_specs=[
            pl.BlockSpec((_BM, _HIDDEN), lambda i: (i, 0)),
            pl.BlockSpec((_EXPERTS, _HIDDEN), lambda i: (0, 0)),
            pl.BlockSpec((1, _EXPERTS), lambda i: (0, 0)),
        ],
        out_specs=(
            pl.BlockSpec((_BM, 2), lambda i: (i, 0)),
            pl.BlockSpec((_BM, 2), lambda i: (i, 0)),
        ),
        out_shape=out_shapes,
    )(hidden_states, W, b2)
    return (idx, mask)


# speedup vs baseline: 1.2282x; 1.2282x over previous
import jax
import jax.numpy as jnp
from jax import lax
from jax.experimental import pallas as pl

_HIDDEN = 2048
_EXPERTS = 16
_THRESH = 0.2
_BM = 2048


def _router_block(h_ref, w_ref, b_ref, idx_ref, mask_ref):
    h = h_ref[...]
    w = w_ref[...]
    logits = lax.dot_general(
        h, w, (((1,), (1,)), ((), ())), preferred_element_type=jnp.float32
    ) + b_ref[...]
    bm = logits.shape[0]
    iota = lax.broadcasted_iota(jnp.int32, (bm, _EXPERTS), 1)
    m1 = jnp.max(logits, axis=1, keepdims=True)
    i1 = jnp.min(jnp.where(logits == m1, iota, _EXPERTS), axis=1, keepdims=True)
    masked = jnp.where(iota == i1, -jnp.inf, logits)
    m2 = jnp.max(masked, axis=1, keepdims=True)
    i2 = jnp.min(jnp.where(masked == m2, iota, _EXPERTS), axis=1, keepdims=True)
    idx_ref[...] = jnp.concatenate([i1, i2], axis=1)
    mask_ref[...] = (jnp.concatenate([m1, m2], axis=1) > _THRESH).astype(jnp.float32)


def kernel(hidden_states, W, b):
    tokens = hidden_states.shape[0]
    grid = (tokens // _BM,)
    b2 = b.reshape(1, _EXPERTS)
    out_shapes = (
        jax.ShapeDtypeStruct((tokens, 2), jnp.int32),
        jax.ShapeDtypeStruct((tokens, 2), jnp.float32),
    )
    idx, mask = pl.pallas_call(
        _router_block,
        grid=grid,
        in_specs=[
            pl.BlockSpec((_BM, _HIDDEN), lambda i: (i, 0)),
            pl.BlockSpec((_EXPERTS, _HIDDEN), lambda i: (0, 0)),
            pl.BlockSpec((1, _EXPERTS), lambda i: (0, 0)),
        ],
        out_specs=(
            pl.BlockSpec((_BM, 2), lambda i: (i, 0)),
            pl.BlockSpec((_BM, 2), lambda i: (i, 0)),
        ),
        out_shape=out_shapes,
    )(hidden_states, W, b2)
    return (idx, mask)


# speedup vs baseline: 1.6828x; 1.3701x over previous
import jax
import jax.numpy as jnp
from jax import lax
from jax.experimental import pallas as pl

_HIDDEN = 2048
_EXPERTS = 16
_THRESH = 0.2
_BM = 2048


def _router_block(h_ref, w_ref, b_ref, idx_ref, mask_ref):
    h = h_ref[...]
    w = w_ref[...]
    logits = lax.dot_general(
        w, h, (((1,), (1,)), ((), ())), preferred_element_type=jnp.float32
    ) + b_ref[...]
    bm = logits.shape[1]
    iota = lax.broadcasted_iota(jnp.int32, (_EXPERTS, bm), 0)
    m1 = jnp.max(logits, axis=0, keepdims=True)
    i1 = jnp.min(jnp.where(logits == m1, iota, _EXPERTS), axis=0, keepdims=True)
    masked = jnp.where(iota == i1, -jnp.inf, logits)
    m2 = jnp.max(masked, axis=0, keepdims=True)
    i2 = jnp.min(jnp.where(masked == m2, iota, _EXPERTS), axis=0, keepdims=True)
    idx_ref[...] = jnp.concatenate([i1, i2], axis=0)
    mask_ref[...] = (jnp.concatenate([m1, m2], axis=0) > _THRESH).astype(jnp.float32)


def kernel(hidden_states, W, b):
    tokens = hidden_states.shape[0]
    grid = (tokens // _BM,)
    b2 = b.reshape(_EXPERTS, 1)
    out_shapes = (
        jax.ShapeDtypeStruct((2, tokens), jnp.int32),
        jax.ShapeDtypeStruct((2, tokens), jnp.float32),
    )
    idx_t, mask_t = pl.pallas_call(
        _router_block,
        grid=grid,
        in_specs=[
            pl.BlockSpec((_BM, _HIDDEN), lambda i: (i, 0)),
            pl.BlockSpec((_EXPERTS, _HIDDEN), lambda i: (0, 0)),
            pl.BlockSpec((_EXPERTS, 1), lambda i: (0, 0)),
        ],
        out_specs=(
            pl.BlockSpec((2, _BM), lambda i: (0, i)),
            pl.BlockSpec((2, _BM), lambda i: (0, i)),
        ),
        out_shape=out_shapes,
    )(hidden_states, W, b2)
    return (idx_t.T, mask_t.T)


# speedup vs baseline: 1.7756x; 1.0552x over previous
import jax
import jax.numpy as jnp
from jax import lax
from jax.experimental import pallas as pl

_HIDDEN = 2048
_EXPERTS = 16
_THRESH = 0.2
_BM = 1024


def _router_block(h_ref, w_ref, b_ref, idx_ref, mask_ref):
    h = h_ref[...]
    w = w_ref[...]
    logits = lax.dot_general(
        w, h, (((1,), (1,)), ((), ())), preferred_element_type=jnp.float32
    ) + b_ref[...]
    bm = logits.shape[1]
    iota = lax.broadcasted_iota(jnp.int32, (_EXPERTS, bm), 0)
    m1 = jnp.max(logits, axis=0, keepdims=True)
    i1 = jnp.min(jnp.where(logits == m1, iota, _EXPERTS), axis=0, keepdims=True)
    masked = jnp.where(iota == i1, -jnp.inf, logits)
    m2 = jnp.max(masked, axis=0, keepdims=True)
    i2 = jnp.min(jnp.where(masked == m2, iota, _EXPERTS), axis=0, keepdims=True)
    idx_ref[...] = jnp.concatenate([i1, i2], axis=0)
    mask_ref[...] = (jnp.concatenate([m1, m2], axis=0) > _THRESH).astype(jnp.float32)


def kernel(hidden_states, W, b):
    tokens = hidden_states.shape[0]
    grid = (tokens // _BM,)
    b2 = b.reshape(_EXPERTS, 1)
    out_shapes = (
        jax.ShapeDtypeStruct((2, tokens), jnp.int32),
        jax.ShapeDtypeStruct((2, tokens), jnp.float32),
    )
    idx_t, mask_t = pl.pallas_call(
        _router_block,
        grid=grid,
        in_specs=[
            pl.BlockSpec((_BM, _HIDDEN), lambda i: (i, 0)),
            pl.BlockSpec((_EXPERTS, _HIDDEN), lambda i: (0, 0)),
            pl.BlockSpec((_EXPERTS, 1), lambda i: (0, 0)),
        ],
        out_specs=(
            pl.BlockSpec((2, _BM), lambda i: (0, i)),
            pl.BlockSpec((2, _BM), lambda i: (0, i)),
        ),
        out_shape=out_shapes,
    )(hidden_states, W, b2)
    return (idx_t.T, mask_t.T)
